# baseline (speedup 1.0000x reference)
import functools

import jax
import jax.numpy as jnp
from jax import lax
from jax.experimental import pallas as pl
from jax.experimental.pallas import tpu as pltpu
from jax.experimental.pallas import tpu_sc as plsc

N = 4096
NH = 2048
D_IN = 768
D_Z = 32
K = 8192
BR = 256
BETA = 0.25

_SC_CORES = 2
_SC_SUBCORES = 16
_BPW = NH // (_SC_CORES * _SC_SUBCORES)


def _dot_nt(a, b):
    return lax.dot_general(a, b, (((1,), (1,)), ((), ())),
                           preferred_element_type=jnp.float32)


def _mlp(h, Ws, bs):
    n = len(Ws)
    for i in range(n):
        h = _dot_nt(h, Ws[i][...]) + bs[i][...]
        if i != n - 1:
            h = jnp.maximum(h, 0.0)
    return h


def _argmin_first(d):
    dmin = jnp.min(d, axis=1, keepdims=True)
    iota = lax.broadcasted_iota(jnp.int32, d.shape, 1)
    return jnp.min(jnp.where(d == dmin, iota, K), axis=1)


def _residual(z, xqs):
    res = z
    for xq in xqs:
        res = res - (res + (xq - res))
    return res


def _dist_argmin(res, cb_ref, cbsq_ref):
    @pl.when(pl.program_id(0) == 0)
    def _():
        cb = cb_ref[...]
        cbsq_ref[...] = jnp.sum(cb * cb, axis=1)[None, :]

    zsq = jnp.sum(res * res, axis=1, keepdims=True)
    d = (zsq + cbsq_ref[...]) - 2.0 * _dot_nt(res, cb_ref[...])
    return _argmin_first(d)


def _enc_body(x_ref, eW0, eW1, eW2, eW3, eb0, eb1, eb2, eb3, cb0_ref,
              z_ref, idx_ref, cbsq_ref):
    z = _mlp(x_ref[...], (eW0, eW1, eW2, eW3), (eb0, eb1, eb2, eb3))
    z_ref[...] = z
    idx_ref[...] = _dist_argmin(z, cb0_ref, cbsq_ref)[:, None]


def _dist_body(n_prev):
    def body(*refs):
        z_ref = refs[0]
        xq_refs = refs[1:1 + n_prev]
        cb_ref = refs[1 + n_prev]
        idx_ref = refs[2 + n_prev]
        cbsq_ref = refs[3 + n_prev]
        res = _residual(z_ref[...], [r[...] for r in xq_refs])
        idx_ref[...] = _dist_argmin(res, cb_ref, cbsq_ref)[:, None]
    return body


def _dec_body(z_ref, xq0_ref, xq1_ref, xq2_ref,
              dW0, dW1, dW2, dW3, db0, db1, db2, db3,
              out_ref, loss_ref):
    z = z_ref[...]
    res = z
    xq_acc = jnp.zeros_like(z)
    total = jnp.float32(0.0)
    for xq_ref in (xq0_ref, xq1_ref, xq2_ref):
        xq = xq_ref[...]
        diff = xq - res
        total = total + jnp.sum(diff * diff)
        xq_st = res + diff
        res = res - xq_st
        xq_acc = xq_acc + xq_st
    out_ref[...] = _mlp(xq_acc, (dW0, dW1, dW2, dW3), (db0, db1, db2, db3))

    @pl.when(pl.program_id(0) == 0)
    def _():
        loss_ref[0, 0] = 0.0

    loss_ref[0, 0] += total


def _full(s):
    return pl.BlockSpec(s, lambda i: (0,) * len(s))


def _rows(w):
    return pl.BlockSpec((BR, w), lambda i: (i, 0))


@functools.partial(
    pl.kernel,
    mesh=plsc.VectorSubcoreMesh(core_axis_name="c", subcore_axis_name="s"),
    out_type=jax.ShapeDtypeStruct((NH, D_Z), jnp.float32),
    scratch_types=[
        pltpu.VMEM((_BPW,), jnp.int32),
        pltpu.VMEM((_BPW, D_Z), jnp.float32),
        pltpu.SemaphoreType.DMA,
    ],
    compiler_params=pltpu.CompilerParams(use_tc_tiling_on_sc=False),
)
def _sc_gather(table_hbm, idx_hbm, out_hbm, idx_v, rows_v, sem):
    wid = lax.axis_index("s") * _SC_CORES + lax.axis_index("c")
    base = wid * _BPW
    pltpu.sync_copy(idx_hbm.at[pl.ds(base, _BPW)], idx_v)
    pltpu.async_copy(table_hbm.at[idx_v], rows_v, sem).wait()
    pltpu.sync_copy(rows_v, out_hbm.at[pl.ds(base, _BPW)])


def _half_chain(x, enc_Ws, enc_bs, dec_Ws, dec_bs, cb0, cb1, cb2):
    z, idx0 = pl.pallas_call(
        _enc_body,
        grid=(NH // BR,),
        in_specs=([_rows(D_IN)]
                  + [_full(w.shape) for w in enc_Ws]
                  + [_full(b.shape) for b in enc_bs]
                  + [_full((K, D_Z))]),
        out_specs=(_rows(D_Z), _rows(1)),
        out_shape=(jax.ShapeDtypeStruct((NH, D_Z), jnp.float32),
                   jax.ShapeDtypeStruct((NH, 1), jnp.int32)),
        scratch_shapes=[pltpu.VMEM((1, K), jnp.float32)],
    )(x, *enc_Ws, *enc_bs, cb0)

    xq0 = _sc_gather(cb0, idx0.reshape(NH))

    idx1 = pl.pallas_call(
        _dist_body(1),
        grid=(NH // BR,),
        in_specs=[_rows(D_Z), _rows(D_Z), _full((K, D_Z))],
        out_specs=_rows(1),
        out_shape=jax.ShapeDtypeStruct((NH, 1), jnp.int32),
        scratch_shapes=[pltpu.VMEM((1, K), jnp.float32)],
    )(z, xq0, cb1)

    xq1 = _sc_gather(cb1, idx1.reshape(NH))

    idx2 = pl.pallas_call(
        _dist_body(2),
        grid=(NH // BR,),
        in_specs=[_rows(D_Z), _rows(D_Z), _rows(D_Z), _full((K, D_Z))],
        out_specs=_rows(1),
        out_shape=jax.ShapeDtypeStruct((NH, 1), jnp.int32),
        scratch_shapes=[pltpu.VMEM((1, K), jnp.float32)],
    )(z, xq0, xq1, cb2)

    xq2 = _sc_gather(cb2, idx2.reshape(NH))

    out, loss = pl.pallas_call(
        _dec_body,
        grid=(NH // BR,),
        in_specs=([_rows(D_Z)] * 4
                  + [_full(w.shape) for w in dec_Ws]
                  + [_full(b.shape) for b in dec_bs]),
        out_specs=(_rows(D_IN), pl.BlockSpec(memory_space=pltpu.SMEM)),
        out_shape=(jax.ShapeDtypeStruct((NH, D_IN), jnp.float32),
                   jax.ShapeDtypeStruct((1, 1), jnp.float32)),
    )(z, xq0, xq1, xq2, *dec_Ws, *dec_bs)

    return out, loss, idx0, idx1, idx2


def kernel(x, enc_W0, enc_b0, enc_W1, enc_b1, enc_W2, enc_b2, enc_W3,
           enc_b3, dec_W0, dec_b0, dec_W1, dec_b1, dec_W2, dec_b2, dec_W3,
           dec_b3, cb0, cb1, cb2):
    enc_Ws = (enc_W0, enc_W1, enc_W2, enc_W3)
    enc_bs = tuple(b[None, :] for b in (enc_b0, enc_b1, enc_b2, enc_b3))
    dec_Ws = (dec_W0, dec_W1, dec_W2, dec_W3)
    dec_bs = tuple(b[None, :] for b in (dec_b0, dec_b1, dec_b2, dec_b3))

    halves = [
        _half_chain(x[h * NH:(h + 1) * NH], enc_Ws, enc_bs, dec_Ws, dec_bs,
                    cb0, cb1, cb2)
        for h in range(2)
    ]

    out = jnp.concatenate([h[0] for h in halves], axis=0)
    loss_sum = halves[0][1][0, 0] + halves[1][1][0, 0]
    rq_loss = loss_sum * jnp.float32((1.0 + BETA) / (3.0 * N * D_Z))
    all_idx = jnp.concatenate(
        [jnp.concatenate([h[2 + s] for h in halves], axis=0)
         for s in range(3)], axis=1)
    return (out, rq_loss, all_idx)


# speedup vs baseline: 1.1978x; 1.1978x over previous
import functools

import jax
import jax.numpy as jnp
from jax import lax
from jax.experimental import pallas as pl
from jax.experimental.pallas import tpu as pltpu
from jax.experimental.pallas import tpu_sc as plsc

N = 4096
D_IN = 768
D_Z = 32
K = 8192
BR = 256
BETA = 0.25

_SC_CORES = 2
_SC_SUBCORES = 16
_BPW = N // (_SC_CORES * _SC_SUBCORES)


def _dot_nt(a, b):
    return lax.dot_general(a, b, (((1,), (1,)), ((), ())),
                           preferred_element_type=jnp.float32)


def _mlp(h, Ws, bs):
    n = len(Ws)
    for i in range(n):
        h = _dot_nt(h, Ws[i][...]) + bs[i][...]
        if i != n - 1:
            h = jnp.maximum(h, 0.0)
    return h


def _argmin_first(d):
    dmin = jnp.min(d, axis=1, keepdims=True)
    iota = lax.broadcasted_iota(jnp.int32, d.shape, 1)
    return jnp.min(jnp.where(d == dmin, iota, K), axis=1)


def _residual(z, xqs):
    res = z
    for xq in xqs:
        res = res - (res + (xq - res))
    return res


def _dist_argmin(res, cb_ref, cbsq_ref):
    @pl.when(pl.program_id(0) == 0)
    def _():
        cb = cb_ref[...]
        cbsq_ref[...] = jnp.sum(cb * cb, axis=1)[None, :]

    zsq = jnp.sum(res * res, axis=1, keepdims=True)
    d = (zsq + cbsq_ref[...]) - 2.0 * _dot_nt(res, cb_ref[...])
    return _argmin_first(d)


def _enc_body(x_ref, eW0, eW1, eW2, eW3, eb0, eb1, eb2, eb3, cb0_ref,
              z_ref, idx_ref, cbsq_ref):
    z = _mlp(x_ref[...], (eW0, eW1, eW2, eW3), (eb0, eb1, eb2, eb3))
    z_ref[...] = z
    idx_ref[...] = _dist_argmin(z, cb0_ref, cbsq_ref)[:, None]


def _dist_body(n_prev):
    def body(*refs):
        z_ref = refs[0]
        xq_refs = refs[1:1 + n_prev]
        cb_ref = refs[1 + n_prev]
        idx_ref = refs[2 + n_prev]
        cbsq_ref = refs[3 + n_prev]
        res = _residual(z_ref[...], [r[...] for r in xq_refs])
        idx_ref[...] = _dist_argmin(res, cb_ref, cbsq_ref)[:, None]
    return body


def _dec_body(z_ref, xq0_ref, xq1_ref, xq2_ref,
              dW0, dW1, dW2, dW3, db0, db1, db2, db3,
              out_ref, loss_ref):
    z = z_ref[...]
    res = z
    xq_acc = jnp.zeros_like(z)
    total = jnp.float32(0.0)
    for xq_ref in (xq0_ref, xq1_ref, xq2_ref):
        xq = xq_ref[...]
        diff = xq - res
        total = total + jnp.sum(diff * diff)
        xq_st = res + diff
        res = res - xq_st
        xq_acc = xq_acc + xq_st
    out_ref[...] = _mlp(xq_acc, (dW0, dW1, dW2, dW3), (db0, db1, db2, db3))

    @pl.when(pl.program_id(0) == 0)
    def _():
        loss_ref[0, 0] = 0.0

    loss_ref[0, 0] += total


def _full(s):
    return pl.BlockSpec(s, lambda i: (0,) * len(s))


def _rows(w):
    return pl.BlockSpec((BR, w), lambda i: (i, 0))


@functools.partial(
    pl.kernel,
    mesh=plsc.VectorSubcoreMesh(core_axis_name="c", subcore_axis_name="s"),
    out_type=jax.ShapeDtypeStruct((N, D_Z), jnp.float32),
    scratch_types=[
        pltpu.VMEM((_BPW,), jnp.int32),
        pltpu.VMEM((_BPW, D_Z), jnp.float32),
        pltpu.SemaphoreType.DMA,
    ],
    compiler_params=pltpu.CompilerParams(use_tc_tiling_on_sc=False),
)
def _sc_gather(table_hbm, idx_hbm, out_hbm, idx_v, rows_v, sem):
    wid = lax.axis_index("s") * _SC_CORES + lax.axis_index("c")
    base = wid * _BPW
    pltpu.sync_copy(idx_hbm.at[pl.ds(base, _BPW)], idx_v)
    pltpu.async_copy(table_hbm.at[idx_v], rows_v, sem).wait()
    pltpu.sync_copy(rows_v, out_hbm.at[pl.ds(base, _BPW)])


def _chain(x, enc_Ws, enc_bs, dec_Ws, dec_bs, cb0, cb1, cb2):
    z, idx0 = pl.pallas_call(
        _enc_body,
        grid=(N // BR,),
        in_specs=([_rows(D_IN)]
                  + [_full(w.shape) for w in enc_Ws]
                  + [_full(b.shape) for b in enc_bs]
                  + [_full((K, D_Z))]),
        out_specs=(_rows(D_Z), _rows(1)),
        out_shape=(jax.ShapeDtypeStruct((N, D_Z), jnp.float32),
                   jax.ShapeDtypeStruct((N, 1), jnp.int32)),
        scratch_shapes=[pltpu.VMEM((1, K), jnp.float32)],
    )(x, *enc_Ws, *enc_bs, cb0)

    xq0 = _sc_gather(cb0, idx0.reshape(N))

    idx1 = pl.pallas_call(
        _dist_body(1),
        grid=(N // BR,),
        in_specs=[_rows(D_Z), _rows(D_Z), _full((K, D_Z))],
        out_specs=_rows(1),
        out_shape=jax.ShapeDtypeStruct((N, 1), jnp.int32),
        scratch_shapes=[pltpu.VMEM((1, K), jnp.float32)],
    )(z, xq0, cb1)

    xq1 = _sc_gather(cb1, idx1.reshape(N))

    idx2 = pl.pallas_call(
        _dist_body(2),
        grid=(N // BR,),
        in_specs=[_rows(D_Z), _rows(D_Z), _rows(D_Z), _full((K, D_Z))],
        out_specs=_rows(1),
        out_shape=jax.ShapeDtypeStruct((N, 1), jnp.int32),
        scratch_shapes=[pltpu.VMEM((1, K), jnp.float32)],
    )(z, xq0, xq1, cb2)

    xq2 = _sc_gather(cb2, idx2.reshape(N))

    out, loss = pl.pallas_call(
        _dec_body,
        grid=(N // BR,),
        in_specs=([_rows(D_Z)] * 4
                  + [_full(w.shape) for w in dec_Ws]
                  + [_full(b.shape) for b in dec_bs]),
        out_specs=(_rows(D_IN), pl.BlockSpec(memory_space=pltpu.SMEM)),
        out_shape=(jax.ShapeDtypeStruct((N, D_IN), jnp.float32),
                   jax.ShapeDtypeStruct((1, 1), jnp.float32)),
    )(z, xq0, xq1, xq2, *dec_Ws, *dec_bs)

    return out, loss, idx0, idx1, idx2


def kernel(x, enc_W0, enc_b0, enc_W1, enc_b1, enc_W2, enc_b2, enc_W3,
           enc_b3, dec_W0, dec_b0, dec_W1, dec_b1, dec_W2, dec_b2, dec_W3,
           dec_b3, cb0, cb1, cb2):
    enc_Ws = (enc_W0, enc_W1, enc_W2, enc_W3)
    enc_bs = tuple(b[None, :] for b in (enc_b0, enc_b1, enc_b2, enc_b3))
    dec_Ws = (dec_W0, dec_W1, dec_W2, dec_W3)
    dec_bs = tuple(b[None, :] for b in (dec_b0, dec_b1, dec_b2, dec_b3))

    out, loss, idx0, idx1, idx2 = _chain(
        x, enc_Ws, enc_bs, dec_Ws, dec_bs, cb0, cb1, cb2)

    rq_loss = loss[0, 0] * jnp.float32((1.0 + BETA) / (3.0 * N * D_Z))
    all_idx = jnp.concatenate([idx0, idx1, idx2], axis=1)
    return (out, rq_loss, all_idx)


# speedup vs baseline: 1.2290x; 1.0260x over previous
import functools

import jax
import jax.numpy as jnp
from jax import lax
from jax.experimental import pallas as pl
from jax.experimental.pallas import tpu as pltpu
from jax.experimental.pallas import tpu_sc as plsc

N = 4096
D_IN = 768
D_Z = 32
K = 8192
BR = 256
BETA = 0.25

_SC_CORES = 2
_SC_SUBCORES = 16
_BPW = N // (_SC_CORES * _SC_SUBCORES)


def _dot_nt(a, b):
    return lax.dot_general(a, b, (((1,), (1,)), ((), ())),
                           preferred_element_type=jnp.float32)


def _mlp(h, Ws, bs):
    n = len(Ws)
    for i in range(n):
        h = _dot_nt(h, Ws[i][...]) + bs[i][...]
        if i != n - 1:
            h = jnp.maximum(h, 0.0)
    return h


def _argmin_first(d):
    dmin = jnp.min(d, axis=1, keepdims=True)
    iota = lax.broadcasted_iota(jnp.int32, d.shape, 1)
    return jnp.min(jnp.where(d == dmin, iota, K), axis=1)


def _residual(z, xqs):
    res = z
    for xq in xqs:
        res = res - (res + (xq - res))
    return res


def _dist_argmin(res, cb_ref, cbsq_ref):
    @pl.when(pl.program_id(0) == 0)
    def _():
        cb = cb_ref[...]
        cbsq_ref[...] = jnp.sum(cb * cb, axis=1)[None, :]

    zsq = jnp.sum(res * res, axis=1, keepdims=True)
    d = (zsq + cbsq_ref[...]) - 2.0 * _dot_nt(res, cb_ref[...])
    return _argmin_first(d)


def _enc_body(x_ref, eW0, eW1, eW2, eW3, eb0, eb1, eb2, eb3, cb0_ref,
              z_ref, idx_ref, cbsq_ref):
    z = _mlp(x_ref[...], (eW0, eW1, eW2, eW3), (eb0, eb1, eb2, eb3))
    z_ref[...] = z
    idx_ref[...] = _dist_argmin(z, cb0_ref, cbsq_ref)


def _dist_body(n_prev):
    def body(*refs):
        z_ref = refs[0]
        xq_refs = refs[1:1 + n_prev]
        cb_ref = refs[1 + n_prev]
        idx_ref = refs[2 + n_prev]
        cbsq_ref = refs[3 + n_prev]
        res = _residual(z_ref[...], [r[...] for r in xq_refs])
        idx_ref[...] = _dist_argmin(res, cb_ref, cbsq_ref)
    return body


def _dec_body(z_ref, xq0_ref, xq1_ref, xq2_ref,
              dW0, dW1, dW2, dW3, db0, db1, db2, db3,
              out_ref, loss_ref):
    z = z_ref[...]
    res = z
    xq_acc = jnp.zeros_like(z)
    total = jnp.float32(0.0)
    for xq_ref in (xq0_ref, xq1_ref, xq2_ref):
        xq = xq_ref[...]
        diff = xq - res
        total = total + jnp.sum(diff * diff)
        xq_st = res + diff
        res = res - xq_st
        xq_acc = xq_acc + xq_st
    out_ref[...] = _mlp(xq_acc, (dW0, dW1, dW2, dW3), (db0, db1, db2, db3))

    @pl.when(pl.program_id(0) == 0)
    def _():
        loss_ref[0, 0] = 0.0

    loss_ref[0, 0] += total


def _full(s):
    return pl.BlockSpec(s, lambda i: (0,) * len(s))


def _rows(w):
    return pl.BlockSpec((BR, w), lambda i: (i, 0))


_IDX_SPEC = pl.BlockSpec((BR,), lambda i: (i,))


@functools.partial(
    pl.kernel,
    mesh=plsc.VectorSubcoreMesh(core_axis_name="c", subcore_axis_name="s"),
    out_type=jax.ShapeDtypeStruct((N, D_Z), jnp.float32),
    scratch_types=[
        pltpu.VMEM((_BPW,), jnp.int32),
        pltpu.VMEM((_BPW, D_Z), jnp.float32),
        pltpu.SemaphoreType.DMA,
    ],
    compiler_params=pltpu.CompilerParams(use_tc_tiling_on_sc=False),
)
def _sc_gather(table_hbm, idx_hbm, out_hbm, idx_v, rows_v, sem):
    wid = lax.axis_index("s") * _SC_CORES + lax.axis_index("c")
    base = wid * _BPW
    pltpu.sync_copy(idx_hbm.at[pl.ds(base, _BPW)], idx_v)
    pltpu.async_copy(table_hbm.at[idx_v], rows_v, sem).wait()
    pltpu.sync_copy(rows_v, out_hbm.at[pl.ds(base, _BPW)])


def _chain(x, enc_Ws, enc_bs, dec_Ws, dec_bs, cb0, cb1, cb2):
    z, idx0 = pl.pallas_call(
        _enc_body,
        grid=(N // BR,),
        in_specs=([_rows(D_IN)]
                  + [_full(w.shape) for w in enc_Ws]
                  + [_full(b.shape) for b in enc_bs]
                  + [_full((K, D_Z))]),
        out_specs=(_rows(D_Z), _IDX_SPEC),
        out_shape=(jax.ShapeDtypeStruct((N, D_Z), jnp.float32),
                   jax.ShapeDtypeStruct((N,), jnp.int32)),
        scratch_shapes=[pltpu.VMEM((1, K), jnp.float32)],
    )(x, *enc_Ws, *enc_bs, cb0)

    xq0 = _sc_gather(cb0, idx0)

    idx1 = pl.pallas_call(
        _dist_body(1),
        grid=(N // BR,),
        in_specs=[_rows(D_Z), _rows(D_Z), _full((K, D_Z))],
        out_specs=_IDX_SPEC,
        out_shape=jax.ShapeDtypeStruct((N,), jnp.int32),
        scratch_shapes=[pltpu.VMEM((1, K), jnp.float32)],
    )(z, xq0, cb1)

    xq1 = _sc_gather(cb1, idx1)

    idx2 = pl.pallas_call(
        _dist_body(2),
        grid=(N // BR,),
        in_specs=[_rows(D_Z), _rows(D_Z), _rows(D_Z), _full((K, D_Z))],
        out_specs=_IDX_SPEC,
        out_shape=jax.ShapeDtypeStruct((N,), jnp.int32),
        scratch_shapes=[pltpu.VMEM((1, K), jnp.float32)],
    )(z, xq0, xq1, cb2)

    xq2 = _sc_gather(cb2, idx2)

    out, loss = pl.pallas_call(
        _dec_body,
        grid=(N // BR,),
        in_specs=([_rows(D_Z)] * 4
                  + [_full(w.shape) for w in dec_Ws]
                  + [_full(b.shape) for b in dec_bs]),
        out_specs=(_rows(D_IN), pl.BlockSpec(memory_space=pltpu.SMEM)),
        out_shape=(jax.ShapeDtypeStruct((N, D_IN), jnp.float32),
                   jax.ShapeDtypeStruct((1, 1), jnp.float32)),
    )(z, xq0, xq1, xq2, *dec_Ws, *dec_bs)

    return out, loss, idx0, idx1, idx2


def kernel(x, enc_W0, enc_b0, enc_W1, enc_b1, enc_W2, enc_b2, enc_W3,
           enc_b3, dec_W0, dec_b0, dec_W1, dec_b1, dec_W2, dec_b2, dec_W3,
           dec_b3, cb0, cb1, cb2):
    enc_Ws = (enc_W0, enc_W1, enc_W2, enc_W3)
    enc_bs = tuple(b[None, :] for b in (enc_b0, enc_b1, enc_b2, enc_b3))
    dec_Ws = (dec_W0, dec_W1, dec_W2, dec_W3)
    dec_bs = tuple(b[None, :] for b in (dec_b0, dec_b1, dec_b2, dec_b3))

    out, loss, idx0, idx1, idx2 = _chain(
        x, enc_Ws, enc_bs, dec_Ws, dec_bs, cb0, cb1, cb2)

    rq_loss = loss[0, 0] * jnp.float32((1.0 + BETA) / (3.0 * N * D_Z))
    all_idx = jnp.stack([idx0, idx1, idx2], axis=1)
    return (out, rq_loss, all_idx)
